# Initial kernel scaffold; baseline (speedup 1.0000x reference)
import jax
import jax.numpy as jnp
from jax.experimental import pallas as pl


def kernel(user_emb, item_emb, W, b, edge_index):
    raise NotImplementedError("write your pallas kernel here")


# speedup vs baseline: 111.2168x; 111.2168x over previous
import functools

import jax
import jax.numpy as jnp
from jax import lax
from jax.experimental import pallas as pl
from jax.experimental.pallas import tpu as pltpu
from jax.experimental.pallas import tpu_sc as plsc

LANE = 128
CH_ROWS = 16
NEG_SLOPE_ = 0.01
PAD_SPREAD = 128


def _sc_mesh():
    return plsc.VectorSubcoreMesh(core_axis_name="c", subcore_axis_name="s")


def _make_deg_kernel(n_half, deg_sp_len, tile_rows):
    n_chunks = tile_rows // CH_ROWS
    z_len = deg_sp_len // 16
    out_tiles = 4
    out_len = n_half // out_tiles

    @functools.partial(
        pl.kernel,
        out_type=(
            jax.ShapeDtypeStruct((deg_sp_len,), jnp.float32),
            jax.ShapeDtypeStruct((deg_sp_len,), jnp.float32),
        ),
        mesh=_sc_mesh(),
        scratch_types=[
            pltpu.VMEM((CH_ROWS, LANE), jnp.int32),
            pltpu.VMEM((LANE,), jnp.float32),
            pltpu.VMEM((z_len,), jnp.float32),
            pltpu.VMEM_SHARED((deg_sp_len,), jnp.float32),
            pltpu.SemaphoreType.DMA,
        ],
    )
    def deg_kernel(scat2_hbm, deg_u_hbm, deg_i_hbm, idx_v, ones_v, zbuf_v,
                   deg_sp, sem):
        c = lax.axis_index("c")
        w = lax.axis_index("s")
        for i in range(LANE // 16):
            ones_v[pl.ds(i * 16, 16)] = jnp.full((16,), 1.0, jnp.float32)

        def zb(i, carry):
            zbuf_v[pl.ds(i * 16, 16)] = jnp.zeros((16,), jnp.float32)
            return carry

        lax.fori_loop(0, z_len // 16, zb, 0)
        pltpu.sync_copy(zbuf_v, deg_sp.at[pl.ds(pl.multiple_of(w * z_len, 8),
                                                z_len)])
        plsc.subcore_barrier()

        col = 1 - c
        base_row = w * tile_rows

        def chunk(i, carry):
            r0 = pl.multiple_of(base_row + i * CH_ROWS, CH_ROWS)
            pltpu.sync_copy(scat2_hbm.at[col, pl.ds(r0, CH_ROWS), :], idx_v)
            cps = [
                pltpu.async_copy(ones_v, deg_sp.at[idx_v.at[j]], sem, add=True)
                for j in range(CH_ROWS)
            ]
            for cp in cps:
                cp.wait()
            return carry

        lax.fori_loop(0, n_chunks, chunk, 0)
        plsc.subcore_barrier()

        zoff = pl.multiple_of(w * z_len, 8)
        pltpu.sync_copy(deg_sp.at[pl.ds(zoff, z_len)], zbuf_v)

        @pl.when(c == 0)
        def _():
            pltpu.sync_copy(zbuf_v, deg_u_hbm.at[pl.ds(zoff, z_len)])

        @pl.when(c == 1)
        def _():
            pltpu.sync_copy(zbuf_v, deg_i_hbm.at[pl.ds(zoff, z_len)])

    return deg_kernel


def _make_msg_kernel(n_half, d, acc_rows, tile_rows):
    ch_rows = 8
    n_chunks = tile_rows // ch_rows
    chunk_e = ch_rows * LANE
    z_rows = acc_rows // 16
    z_full, z_tail = z_rows // chunk_e, z_rows % chunk_e
    out_rows = -(-n_half // 16 // 16) * 16
    out_last = n_half - 15 * out_rows

    @functools.partial(
        pl.kernel,
        out_type=jax.ShapeDtypeStruct((2 * n_half, d), jnp.float32),
        mesh=_sc_mesh(),
        compiler_params=pltpu.CompilerParams(use_tc_tiling_on_sc=False),
        scratch_types=[
            pltpu.VMEM((ch_rows, LANE), jnp.int32),
            pltpu.VMEM((ch_rows, LANE), jnp.int32),
            pltpu.VMEM((chunk_e, d), jnp.float32),
            pltpu.VMEM_SHARED((acc_rows, d), jnp.float32),
            pltpu.SemaphoreType.DMA,
            pltpu.SemaphoreType.DMA,
        ],
    )
    def msg_kernel(gath2, scat2, y_hbm, t_hbm, gidx_v, sidx_v, rows_v,
                   acc_sp, gsem, ssem):
        c = lax.axis_index("c")
        w = lax.axis_index("s")

        def zr(i, carry):
            rows_v[i, :] = jnp.zeros((d,), jnp.float32)
            return carry

        lax.fori_loop(0, chunk_e, zr, 0)
        zbase = pl.multiple_of(w * z_rows, 16)
        for k in range(z_full):
            pltpu.sync_copy(rows_v, acc_sp.at[pl.ds(zbase + k * chunk_e,
                                                    chunk_e), :])
        if z_tail:
            pltpu.sync_copy(
                rows_v.at[pl.ds(0, z_tail), :],
                acc_sp.at[pl.ds(zbase + z_full * chunk_e, z_tail), :])
        plsc.subcore_barrier()

        base_row = w * tile_rows

        def chunk(i, carry):
            r0 = pl.multiple_of(base_row + i * ch_rows, ch_rows)
            pltpu.sync_copy(gath2.at[c, pl.ds(r0, ch_rows), :], gidx_v)
            pltpu.sync_copy(scat2.at[c, pl.ds(r0, ch_rows), :], sidx_v)
            gs = [
                pltpu.async_copy(
                    y_hbm.at[gidx_v.at[j]],
                    rows_v.at[pl.ds(j * LANE, LANE), :], gsem)
                for j in range(ch_rows)
            ]
            for cp in gs:
                cp.wait()
            ss = [
                pltpu.async_copy(
                    rows_v.at[pl.ds(j * LANE, LANE), :],
                    acc_sp.at[sidx_v.at[j]], ssem, add=True)
                for j in range(ch_rows)
            ]
            for cp in ss:
                cp.wait()
            return carry

        lax.fori_loop(0, n_chunks, chunk, 0)
        plsc.subcore_barrier()

        src_base = pl.multiple_of(w * out_rows, 16)
        dst_base = pl.multiple_of((1 - c) * n_half + w * out_rows, 16)
        o_full, o_tail = out_rows // chunk_e, out_rows % chunk_e
        l_full, l_tail = out_last // chunk_e, out_last % chunk_e

        def _bounce(n_f, n_t):
            for k in range(n_f):
                pltpu.sync_copy(
                    acc_sp.at[pl.ds(src_base + k * chunk_e, chunk_e), :],
                    rows_v)
                pltpu.sync_copy(
                    rows_v, t_hbm.at[pl.ds(dst_base + k * chunk_e, chunk_e), :])
            if n_t:
                pltpu.sync_copy(
                    acc_sp.at[pl.ds(src_base + n_f * chunk_e, n_t), :],
                    rows_v.at[pl.ds(0, n_t), :])
                pltpu.sync_copy(
                    rows_v.at[pl.ds(0, n_t), :],
                    t_hbm.at[pl.ds(dst_base + n_f * chunk_e, n_t), :])

        @pl.when(w < 15)
        def _():
            _bounce(o_full, o_tail)

        @pl.when(w == 15)
        def _():
            _bounce(l_full, l_tail)

    return msg_kernel


def _scale_body(x_ref, deg_ref, y_ref):
    degf = deg_ref[...] + 1.0
    y_ref[...] = x_ref[...] * lax.rsqrt(degf)


def _final_body(x_ref, t_ref, deg_ref, w_ref, b_ref, o_ref):
    degf = deg_ref[...] + 1.0
    xv = x_ref[...]
    agg = t_ref[...] * lax.rsqrt(degf) + xv / degf
    z = jnp.dot(agg, w_ref[...], preferred_element_type=jnp.float32) + b_ref[...]
    h = jnp.where(z > 0, z, z * NEG_SLOPE_)
    o_ref[...] = 0.5 * (xv + h)


def kernel(user_emb, item_emb, W, b, edge_index):
    n_half, d = user_emb.shape
    assert item_emb.shape[0] == n_half
    n = 2 * n_half
    e = edge_index.shape[1]

    rows = -(-e // LANE)
    tile_rows = -(-rows // (16 * CH_ROWS)) * CH_ROWS
    r_pad = tile_rows * 16
    e_pad = r_pad * LANE
    npad = e_pad - e

    src = edge_index[0]
    dst = edge_index[1]
    pad_low = jnp.arange(npad, dtype=jnp.int32) % PAD_SPREAD
    pad_high = pad_low + n_half
    g_a = jnp.concatenate([src, pad_high]).reshape(r_pad, LANE)
    g_b = jnp.concatenate([dst + n_half, pad_low]).reshape(r_pad, LANE)
    s_a = jnp.concatenate([dst, pad_high]).reshape(r_pad, LANE)
    s_b = jnp.concatenate([src, pad_high]).reshape(r_pad, LANE)
    gath2 = jnp.stack([g_a, g_b])
    scat2 = jnp.stack([s_a, s_b])

    x = jnp.concatenate([user_emb, item_emb], axis=0)

    deg_sp_len = -(-(n_half + PAD_SPREAD) // 256) * 256
    deg_u, deg_i = _make_deg_kernel(n_half, deg_sp_len, tile_rows)(scat2)
    deg = jnp.concatenate([deg_u[:n_half], deg_i[:n_half]])[:, None]

    blk = 2000
    n_blk = n // blk
    y = pl.pallas_call(
        _scale_body,
        grid=(n_blk,),
        in_specs=[
            pl.BlockSpec((blk, d), lambda i: (i, 0)),
            pl.BlockSpec((blk, 1), lambda i: (i, 0)),
        ],
        out_specs=pl.BlockSpec((blk, d), lambda i: (i, 0)),
        out_shape=jax.ShapeDtypeStruct((n, d), jnp.float32),
    )(x, deg)

    acc_rows = n_half + 2 * PAD_SPREAD + 96
    t = _make_msg_kernel(n_half, d, acc_rows, tile_rows)(gath2, scat2, y)

    out = pl.pallas_call(
        _final_body,
        grid=(n_blk,),
        in_specs=[
            pl.BlockSpec((blk, d), lambda i: (i, 0)),
            pl.BlockSpec((blk, d), lambda i: (i, 0)),
            pl.BlockSpec((blk, 1), lambda i: (i, 0)),
            pl.BlockSpec((d, d), lambda i: (0, 0)),
            pl.BlockSpec((1, d), lambda i: (0, 0)),
        ],
        out_specs=pl.BlockSpec((blk, d), lambda i: (i, 0)),
        out_shape=jax.ShapeDtypeStruct((n, d), jnp.float32),
    )(x, t, deg, W, b)
    return out


# speedup vs baseline: 173.4966x; 1.5600x over previous
import functools

import jax
import jax.numpy as jnp
from jax import lax
from jax.experimental import pallas as pl
from jax.experimental.pallas import tpu as pltpu
from jax.experimental.pallas import tpu_sc as plsc

LANE = 128
CH_ROWS = 16
NEG_SLOPE_ = 0.01
PAD_SPREAD = 128


def _sc_mesh():
    return plsc.VectorSubcoreMesh(core_axis_name="c", subcore_axis_name="s")


def _make_deg_kernel(n_half, deg_sp_len, tile_rows):
    n_chunks = tile_rows // CH_ROWS
    z_len = deg_sp_len // 16

    @functools.partial(
        pl.kernel,
        out_type=(
            jax.ShapeDtypeStruct((deg_sp_len,), jnp.float32),
            jax.ShapeDtypeStruct((deg_sp_len,), jnp.float32),
        ),
        mesh=_sc_mesh(),
        scratch_types=[
            pltpu.VMEM((CH_ROWS, LANE), jnp.int32),
            pltpu.VMEM((LANE,), jnp.float32),
            pltpu.VMEM((z_len,), jnp.float32),
            pltpu.VMEM_SHARED((deg_sp_len,), jnp.float32),
            pltpu.SemaphoreType.DMA,
        ],
    )
    def deg_kernel(idx_a_hbm, idx_b_hbm, deg_u_hbm, deg_i_hbm, idx_v, ones_v,
                   zbuf_v, deg_sp, sem):
        c = lax.axis_index("c")
        w = lax.axis_index("s")
        for i in range(LANE // 16):
            ones_v[pl.ds(i * 16, 16)] = jnp.full((16,), 1.0, jnp.float32)

        def zb(i, carry):
            zbuf_v[pl.ds(i * 16, 16)] = jnp.zeros((16,), jnp.float32)
            return carry

        lax.fori_loop(0, z_len // 16, zb, 0)
        pltpu.sync_copy(zbuf_v, deg_sp.at[pl.ds(pl.multiple_of(w * z_len, 8),
                                                z_len)])
        plsc.subcore_barrier()

        base_row = w * tile_rows

        def chunk(i, carry):
            r0 = pl.multiple_of(base_row + i * CH_ROWS, CH_ROWS)

            @pl.when(c == 0)
            def _():
                pltpu.sync_copy(idx_a_hbm.at[pl.ds(r0, CH_ROWS), :], idx_v)

            @pl.when(c == 1)
            def _():
                pltpu.sync_copy(idx_b_hbm.at[pl.ds(r0, CH_ROWS), :], idx_v)

            cps = [
                pltpu.async_copy(ones_v, deg_sp.at[idx_v.at[j]], sem, add=True)
                for j in range(CH_ROWS)
            ]
            for cp in cps:
                cp.wait()
            return carry

        lax.fori_loop(0, n_chunks, chunk, 0)
        plsc.subcore_barrier()

        zoff = pl.multiple_of(w * z_len, 8)
        pltpu.sync_copy(deg_sp.at[pl.ds(zoff, z_len)], zbuf_v)

        @pl.when(c == 0)
        def _():
            pltpu.sync_copy(zbuf_v, deg_u_hbm.at[pl.ds(zoff, z_len)])

        @pl.when(c == 1)
        def _():
            pltpu.sync_copy(zbuf_v, deg_i_hbm.at[pl.ds(zoff, z_len)])

    return deg_kernel


def _make_msg_kernel(n_half, d, acc_rows, tile_rows):
    ch_rows = 8
    n_chunks = tile_rows // ch_rows
    chunk_e = ch_rows * LANE
    z_rows = acc_rows // 16
    z_full, z_tail = z_rows // chunk_e, z_rows % chunk_e
    out_rows = -(-n_half // 16 // 16) * 16
    out_last = n_half - 15 * out_rows

    @functools.partial(
        pl.kernel,
        out_type=jax.ShapeDtypeStruct((2 * n_half, d), jnp.float32),
        mesh=_sc_mesh(),
        compiler_params=pltpu.CompilerParams(use_tc_tiling_on_sc=False),
        scratch_types=[
            pltpu.VMEM((ch_rows, LANE), jnp.int32),
            pltpu.VMEM((ch_rows, LANE), jnp.int32),
            pltpu.VMEM((chunk_e, d), jnp.float32),
            pltpu.VMEM_SHARED((acc_rows, d), jnp.float32),
            pltpu.SemaphoreType.DMA,
            pltpu.SemaphoreType.DMA,
        ],
    )
    def msg_kernel(idx_a_hbm, idx_b_hbm, idx_b2_hbm, y_hbm, t_hbm,
                   gidx_v, sidx_v, rows_v, acc_sp, gsem, ssem):
        c = lax.axis_index("c")
        w = lax.axis_index("s")

        def zr(i, carry):
            rows_v[i, :] = jnp.zeros((d,), jnp.float32)
            return carry

        lax.fori_loop(0, chunk_e, zr, 0)
        zbase = pl.multiple_of(w * z_rows, 16)
        for k in range(z_full):
            pltpu.sync_copy(rows_v, acc_sp.at[pl.ds(zbase + k * chunk_e,
                                                    chunk_e), :])
        if z_tail:
            pltpu.sync_copy(
                rows_v.at[pl.ds(0, z_tail), :],
                acc_sp.at[pl.ds(zbase + z_full * chunk_e, z_tail), :])
        plsc.subcore_barrier()

        base_row = w * tile_rows

        def chunk(i, carry):
            r0 = pl.multiple_of(base_row + i * ch_rows, ch_rows)

            @pl.when(c == 0)
            def _():
                pltpu.sync_copy(idx_a_hbm.at[pl.ds(r0, ch_rows), :], gidx_v)
                pltpu.sync_copy(idx_b_hbm.at[pl.ds(r0, ch_rows), :], sidx_v)

            @pl.when(c == 1)
            def _():
                pltpu.sync_copy(idx_b2_hbm.at[pl.ds(r0, ch_rows), :], gidx_v)
                pltpu.sync_copy(idx_a_hbm.at[pl.ds(r0, ch_rows), :], sidx_v)

            gs = [
                pltpu.async_copy(
                    y_hbm.at[gidx_v.at[j]],
                    rows_v.at[pl.ds(j * LANE, LANE), :], gsem)
                for j in range(ch_rows)
            ]
            for cp in gs:
                cp.wait()
            ss = [
                pltpu.async_copy(
                    rows_v.at[pl.ds(j * LANE, LANE), :],
                    acc_sp.at[sidx_v.at[j]], ssem, add=True)
                for j in range(ch_rows)
            ]
            for cp in ss:
                cp.wait()
            return carry

        lax.fori_loop(0, n_chunks, chunk, 0)
        plsc.subcore_barrier()

        src_base = pl.multiple_of(w * out_rows, 16)
        dst_base = pl.multiple_of((1 - c) * n_half + w * out_rows, 16)
        o_full, o_tail = out_rows // chunk_e, out_rows % chunk_e
        l_full, l_tail = out_last // chunk_e, out_last % chunk_e

        def _bounce(n_f, n_t):
            for k in range(n_f):
                pltpu.sync_copy(
                    acc_sp.at[pl.ds(src_base + k * chunk_e, chunk_e), :],
                    rows_v)
                pltpu.sync_copy(
                    rows_v, t_hbm.at[pl.ds(dst_base + k * chunk_e, chunk_e), :])
            if n_t:
                pltpu.sync_copy(
                    acc_sp.at[pl.ds(src_base + n_f * chunk_e, n_t), :],
                    rows_v.at[pl.ds(0, n_t), :])
                pltpu.sync_copy(
                    rows_v.at[pl.ds(0, n_t), :],
                    t_hbm.at[pl.ds(dst_base + n_f * chunk_e, n_t), :])

        @pl.when(w < 15)
        def _():
            _bounce(o_full, o_tail)

        @pl.when(w == 15)
        def _():
            _bounce(l_full, l_tail)

    return msg_kernel


def _scale_body(x_ref, deg8_ref, r8_ref, y_ref):
    d128 = jnp.dot(deg8_ref[...], r8_ref[...],
                   preferred_element_type=jnp.float32) + 1.0
    y_ref[...] = x_ref[...] * lax.rsqrt(d128)


def _final_body(x_ref, t_ref, deg8_ref, r8_ref, wbig_ref, bbig_ref, o_ref):
    d128 = jnp.dot(deg8_ref[...], r8_ref[...],
                   preferred_element_type=jnp.float32) + 1.0
    xv = x_ref[...]
    agg = t_ref[...] * lax.rsqrt(d128) + xv / d128
    z = jnp.dot(agg, wbig_ref[...],
                preferred_element_type=jnp.float32) + bbig_ref[...]
    h = jnp.where(z > 0, z, z * NEG_SLOPE_)
    o_ref[...] = 0.5 * (xv + h)


def kernel(user_emb, item_emb, W, b, edge_index):
    n_half, d = user_emb.shape
    assert item_emb.shape[0] == n_half
    n = 2 * n_half
    e = edge_index.shape[1]
    per_row = LANE // d

    rows = -(-e // LANE)
    tile_rows = -(-rows // (16 * CH_ROWS)) * CH_ROWS
    r_pad = tile_rows * 16
    e_pad = r_pad * LANE
    npad = e_pad - e

    src = edge_index[0]
    dst = edge_index[1]
    pad_h = jnp.arange(npad, dtype=jnp.int32) % PAD_SPREAD + n_half
    idx_a = jnp.concatenate([src, pad_h]).reshape(r_pad, LANE)
    idx_b = jnp.concatenate([dst, pad_h]).reshape(r_pad, LANE)
    idx_b2 = jnp.concatenate([dst + n_half, pad_h]).reshape(r_pad, LANE)

    deg_sp_len = -(-(n_half + PAD_SPREAD) // 256) * 256
    deg_u, deg_i = _make_deg_kernel(n_half, deg_sp_len, tile_rows)(idx_a,
                                                                   idx_b)
    deg8 = jnp.concatenate([deg_u[:n_half], deg_i[:n_half]]).reshape(
        n // per_row, per_row)

    xf = jnp.concatenate([user_emb, item_emb]).reshape(n // per_row, LANE)
    r8 = jnp.repeat(jnp.eye(per_row, dtype=jnp.float32), d, axis=1)
    blk = 5000
    n_blk = (n // per_row) // blk
    yf = pl.pallas_call(
        _scale_body,
        grid=(n_blk,),
        in_specs=[
            pl.BlockSpec((blk, LANE), lambda i: (i, 0)),
            pl.BlockSpec((blk, per_row), lambda i: (i, 0)),
            pl.BlockSpec((per_row, LANE), lambda i: (0, 0)),
        ],
        out_specs=pl.BlockSpec((blk, LANE), lambda i: (i, 0)),
        out_shape=jax.ShapeDtypeStruct((n // per_row, LANE), jnp.float32),
    )(xf, deg8, r8)

    acc_rows = deg_sp_len
    y = yf.reshape(n, d)
    t = _make_msg_kernel(n_half, d, acc_rows, tile_rows)(idx_a, idx_b,
                                                         idx_b2, y)

    tf = t.reshape(n // per_row, LANE)
    w_big = jnp.kron(jnp.eye(per_row, dtype=jnp.float32), W)
    b_big = jnp.tile(b.reshape(-1), per_row)[None, :]
    of = pl.pallas_call(
        _final_body,
        grid=(n_blk,),
        in_specs=[
            pl.BlockSpec((blk, LANE), lambda i: (i, 0)),
            pl.BlockSpec((blk, LANE), lambda i: (i, 0)),
            pl.BlockSpec((blk, per_row), lambda i: (i, 0)),
            pl.BlockSpec((per_row, LANE), lambda i: (0, 0)),
            pl.BlockSpec((LANE, LANE), lambda i: (0, 0)),
            pl.BlockSpec((1, LANE), lambda i: (0, 0)),
        ],
        out_specs=pl.BlockSpec((blk, LANE), lambda i: (i, 0)),
        out_shape=jax.ShapeDtypeStruct((n // per_row, LANE), jnp.float32),
    )(xf, tf, deg8, r8, w_big, b_big)
    return of.reshape(n, d)


# speedup vs baseline: 182.2212x; 1.0503x over previous
import functools

import jax
import jax.numpy as jnp
from jax import lax
from jax.experimental import pallas as pl
from jax.experimental.pallas import tpu as pltpu
from jax.experimental.pallas import tpu_sc as plsc

LANE = 128
CH_ROWS = 16
NEG_SLOPE_ = 0.01
PAD_SPREAD = 128


def _sc_mesh():
    return plsc.VectorSubcoreMesh(core_axis_name="c", subcore_axis_name="s")


def _make_deg_kernel(n_half, deg_sp_len, tile_rows, lt_rows):
    n_chunks = tile_rows // CH_ROWS
    lt_full, lt_part = lt_rows // CH_ROWS, lt_rows % CH_ROWS
    z_len = deg_sp_len // 16
    z_last = n_half - 15 * z_len

    @functools.partial(
        pl.kernel,
        out_type=jax.ShapeDtypeStruct((2 * n_half,), jnp.float32),
        mesh=_sc_mesh(),
        compiler_params=pltpu.CompilerParams(use_tc_tiling_on_sc=False),
        scratch_types=[
            pltpu.VMEM((CH_ROWS, LANE), jnp.int32),
            pltpu.VMEM((LANE,), jnp.float32),
            pltpu.VMEM((z_len,), jnp.float32),
            pltpu.VMEM_SHARED((deg_sp_len,), jnp.float32),
            pltpu.SemaphoreType.DMA,
        ],
    )
    def deg_kernel(idx_a_hbm, idx_b_hbm, deg_hbm, idx_v, ones_v,
                   zbuf_v, deg_sp, sem):
        c = lax.axis_index("c")
        w = lax.axis_index("s")
        for i in range(LANE // 16):
            ones_v[pl.ds(i * 16, 16)] = jnp.full((16,), 1.0, jnp.float32)

        def zb(i, carry):
            zbuf_v[pl.ds(i * 16, 16)] = jnp.zeros((16,), jnp.float32)
            return carry

        lax.fori_loop(0, z_len // 16, zb, 0)
        pltpu.sync_copy(zbuf_v, deg_sp.at[pl.ds(pl.multiple_of(w * z_len, 8),
                                                z_len)])
        plsc.subcore_barrier()

        base_row = w * tile_rows

        def do_chunk(r0, nrows):
            @pl.when(c == 0)
            def _():
                pltpu.sync_copy(idx_a_hbm.at[pl.ds(r0, nrows), :],
                                idx_v.at[pl.ds(0, nrows), :])

            @pl.when(c == 1)
            def _():
                pltpu.sync_copy(idx_b_hbm.at[pl.ds(r0, nrows), :],
                                idx_v.at[pl.ds(0, nrows), :])

            cps = [
                pltpu.async_copy(ones_v, deg_sp.at[idx_v.at[j]], sem, add=True)
                for j in range(nrows)
            ]
            for cp in cps:
                cp.wait()

        def chunk(i, carry):
            do_chunk(base_row + i * CH_ROWS, CH_ROWS)
            return carry

        n_ch = jnp.where(w == 15, lt_full, n_chunks)
        lax.fori_loop(0, n_ch, chunk, 0)
        if lt_part:
            @pl.when(w == 15)
            def _():
                do_chunk(base_row + lt_full * CH_ROWS, lt_part)
        plsc.subcore_barrier()

        zoff = w * z_len

        @pl.when(w < 15)
        def _():
            pltpu.sync_copy(deg_sp.at[pl.ds(zoff, z_len)], zbuf_v)
            pltpu.sync_copy(zbuf_v,
                            deg_hbm.at[pl.ds(c * n_half + zoff, z_len)])

        @pl.when(w == 15)
        def _():
            pltpu.sync_copy(deg_sp.at[pl.ds(zoff, z_last)],
                            zbuf_v.at[pl.ds(0, z_last)])
            pltpu.sync_copy(zbuf_v.at[pl.ds(0, z_last)],
                            deg_hbm.at[pl.ds(c * n_half + zoff, z_last)])

    return deg_kernel


def _make_msg_kernel(n_half, d, acc_rows, tile_rows, lt_rows):
    ch_rows = 8
    n_chunks = tile_rows // ch_rows
    lt_full, lt_part = lt_rows // ch_rows, lt_rows % ch_rows
    chunk_e = ch_rows * LANE
    z_rows = acc_rows // 16
    z_full, z_tail = z_rows // chunk_e, z_rows % chunk_e
    out_rows = -(-n_half // 16 // 16) * 16
    out_last = n_half - 15 * out_rows

    @functools.partial(
        pl.kernel,
        out_type=jax.ShapeDtypeStruct((2 * n_half, d), jnp.float32),
        mesh=_sc_mesh(),
        compiler_params=pltpu.CompilerParams(use_tc_tiling_on_sc=False),
        scratch_types=[
            pltpu.VMEM((ch_rows, LANE), jnp.int32),
            pltpu.VMEM((ch_rows, LANE), jnp.int32),
            pltpu.VMEM((chunk_e, d), jnp.float32),
            pltpu.VMEM_SHARED((acc_rows, d), jnp.float32),
            pltpu.SemaphoreType.DMA,
            pltpu.SemaphoreType.DMA,
        ],
    )
    def msg_kernel(idx_a_hbm, idx_b_hbm, idx_b2_hbm, y_hbm, t_hbm,
                   gidx_v, sidx_v, rows_v, acc_sp, gsem, ssem):
        c = lax.axis_index("c")
        w = lax.axis_index("s")

        def zr(i, carry):
            rows_v[i, :] = jnp.zeros((d,), jnp.float32)
            return carry

        lax.fori_loop(0, chunk_e, zr, 0)
        zbase = pl.multiple_of(w * z_rows, 16)
        for k in range(z_full):
            pltpu.sync_copy(rows_v, acc_sp.at[pl.ds(zbase + k * chunk_e,
                                                    chunk_e), :])
        if z_tail:
            pltpu.sync_copy(
                rows_v.at[pl.ds(0, z_tail), :],
                acc_sp.at[pl.ds(zbase + z_full * chunk_e, z_tail), :])
        plsc.subcore_barrier()

        base_row = w * tile_rows

        def do_chunk(r0, nrows):
            @pl.when(c == 0)
            def _():
                pltpu.sync_copy(idx_a_hbm.at[pl.ds(r0, nrows), :],
                                gidx_v.at[pl.ds(0, nrows), :])
                pltpu.sync_copy(idx_b_hbm.at[pl.ds(r0, nrows), :],
                                sidx_v.at[pl.ds(0, nrows), :])

            @pl.when(c == 1)
            def _():
                pltpu.sync_copy(idx_b2_hbm.at[pl.ds(r0, nrows), :],
                                gidx_v.at[pl.ds(0, nrows), :])
                pltpu.sync_copy(idx_a_hbm.at[pl.ds(r0, nrows), :],
                                sidx_v.at[pl.ds(0, nrows), :])

            gs = [
                pltpu.async_copy(
                    y_hbm.at[gidx_v.at[j]],
                    rows_v.at[pl.ds(j * LANE, LANE), :], gsem)
                for j in range(nrows)
            ]
            for cp in gs:
                cp.wait()
            ss = [
                pltpu.async_copy(
                    rows_v.at[pl.ds(j * LANE, LANE), :],
                    acc_sp.at[sidx_v.at[j]], ssem, add=True)
                for j in range(nrows)
            ]
            for cp in ss:
                cp.wait()

        def chunk(i, carry):
            do_chunk(base_row + i * ch_rows, ch_rows)
            return carry

        n_ch = jnp.where(w == 15, lt_full, n_chunks)
        lax.fori_loop(0, n_ch, chunk, 0)
        if lt_part:
            @pl.when(w == 15)
            def _():
                do_chunk(base_row + lt_full * ch_rows, lt_part)
        plsc.subcore_barrier()

        src_base = pl.multiple_of(w * out_rows, 16)
        dst_base = pl.multiple_of((1 - c) * n_half + w * out_rows, 16)
        o_full, o_tail = out_rows // chunk_e, out_rows % chunk_e
        l_full, l_tail = out_last // chunk_e, out_last % chunk_e

        def _bounce(n_f, n_t):
            for k in range(n_f):
                pltpu.sync_copy(
                    acc_sp.at[pl.ds(src_base + k * chunk_e, chunk_e), :],
                    rows_v)
                pltpu.sync_copy(
                    rows_v, t_hbm.at[pl.ds(dst_base + k * chunk_e, chunk_e), :])
            if n_t:
                pltpu.sync_copy(
                    acc_sp.at[pl.ds(src_base + n_f * chunk_e, n_t), :],
                    rows_v.at[pl.ds(0, n_t), :])
                pltpu.sync_copy(
                    rows_v.at[pl.ds(0, n_t), :],
                    t_hbm.at[pl.ds(dst_base + n_f * chunk_e, n_t), :])

        @pl.when(w < 15)
        def _():
            _bounce(o_full, o_tail)

        @pl.when(w == 15)
        def _():
            _bounce(l_full, l_tail)

    return msg_kernel


def _scale_body(x_ref, deg8_ref, r8_ref, y_ref):
    d128 = jnp.dot(deg8_ref[...], r8_ref[...],
                   preferred_element_type=jnp.float32) + 1.0
    y_ref[...] = x_ref[...] * lax.rsqrt(d128)


def _final_body(x_ref, t_ref, deg8_ref, r8_ref, wbig_ref, bbig_ref, o_ref):
    d128 = jnp.dot(deg8_ref[...], r8_ref[...],
                   preferred_element_type=jnp.float32) + 1.0
    xv = x_ref[...]
    agg = t_ref[...] * lax.rsqrt(d128) + xv / d128
    z = jnp.dot(agg, wbig_ref[...],
                preferred_element_type=jnp.float32) + bbig_ref[...]
    h = jnp.where(z > 0, z, z * NEG_SLOPE_)
    o_ref[...] = 0.5 * (xv + h)


def kernel(user_emb, item_emb, W, b, edge_index):
    n_half, d = user_emb.shape
    assert item_emb.shape[0] == n_half
    n = 2 * n_half
    e = edge_index.shape[1]
    per_row = LANE // d
    assert e % LANE == 0

    rows = e // LANE
    tile_rows = -(-rows // (16 * CH_ROWS)) * CH_ROWS
    lt_rows = rows - 15 * tile_rows

    idx_a = edge_index[0].reshape(rows, LANE)
    idx_b = edge_index[1].reshape(rows, LANE)
    idx_b2 = (edge_index[1] + n_half).reshape(rows, LANE)

    deg_sp_len = -(-(n_half + PAD_SPREAD) // 256) * 256
    deg_cat = _make_deg_kernel(n_half, deg_sp_len, tile_rows,
                               lt_rows)(idx_a, idx_b)
    deg8 = deg_cat.reshape(n // per_row, per_row)

    xf = jnp.concatenate([user_emb, item_emb]).reshape(n // per_row, LANE)
    r8 = jnp.repeat(jnp.eye(per_row, dtype=jnp.float32), d, axis=1)
    blk = 5000
    n_blk = (n // per_row) // blk
    yf = pl.pallas_call(
        _scale_body,
        grid=(n_blk,),
        in_specs=[
            pl.BlockSpec((blk, LANE), lambda i: (i, 0)),
            pl.BlockSpec((blk, per_row), lambda i: (i, 0)),
            pl.BlockSpec((per_row, LANE), lambda i: (0, 0)),
        ],
        out_specs=pl.BlockSpec((blk, LANE), lambda i: (i, 0)),
        out_shape=jax.ShapeDtypeStruct((n // per_row, LANE), jnp.float32),
    )(xf, deg8, r8)

    acc_rows = deg_sp_len
    y = yf.reshape(n, d)
    t = _make_msg_kernel(n_half, d, acc_rows, tile_rows,
                         lt_rows)(idx_a, idx_b, idx_b2, y)

    tf = t.reshape(n // per_row, LANE)
    w_big = jnp.kron(jnp.eye(per_row, dtype=jnp.float32), W)
    b_big = jnp.tile(b.reshape(-1), per_row)[None, :]
    of = pl.pallas_call(
        _final_body,
        grid=(n_blk,),
        in_specs=[
            pl.BlockSpec((blk, LANE), lambda i: (i, 0)),
            pl.BlockSpec((blk, LANE), lambda i: (i, 0)),
            pl.BlockSpec((blk, per_row), lambda i: (i, 0)),
            pl.BlockSpec((per_row, LANE), lambda i: (0, 0)),
            pl.BlockSpec((LANE, LANE), lambda i: (0, 0)),
            pl.BlockSpec((1, LANE), lambda i: (0, 0)),
        ],
        out_specs=pl.BlockSpec((blk, LANE), lambda i: (i, 0)),
        out_shape=jax.ShapeDtypeStruct((n // per_row, LANE), jnp.float32),
    )(xf, tf, deg8, r8, w_big, b_big)
    return of.reshape(n, d)
